# Initial kernel scaffold; baseline (speedup 1.0000x reference)
import functools

import jax
import jax.numpy as jnp
from jax import lax
from jax.experimental import pallas as pl
from jax.experimental.pallas import tpu as pltpu
from jax.experimental.pallas import tpu_sc as plsc

N_CENTERS = 10000
CAP = 100
FEAT = 64
BATCH = 16384


CB = 100
GRID = N_CENTERS // CB


def _means_body(cnt_ref, bank_ref, out_ref):
    cnt = cnt_ref[0, 0, :]
    rows = bank_ref[...]
    pos = lax.broadcasted_iota(jnp.int32, (CB, CAP, FEAT), 1)
    mask = (pos < cnt[:, None, None]).astype(jnp.float32)
    sums = jnp.sum(rows * mask, axis=1)
    denom = jnp.maximum(cnt, 1).astype(jnp.float32)[:, None]
    out_ref[...] = sums / denom


def _center_means(count, bank):
    cnt3 = count.reshape(GRID, 1, CB)
    return pl.pallas_call(
        _means_body,
        grid=(GRID,),
        in_specs=[
            pl.BlockSpec((1, 1, CB), lambda i: (i, 0, 0)),
            pl.BlockSpec((CB, CAP, FEAT), lambda i: (i, 0, 0)),
        ],
        out_specs=pl.BlockSpec((CB, FEAT), lambda i: (i, 0)),
        out_shape=jax.ShapeDtypeStruct((N_CENTERS, FEAT), jnp.float32),
    )(cnt3, bank)


NC = 2
NS = 16
NW = NC * NS
B_PER_W = BATCH // NW
CHUNKS = B_PER_W // 16

_MESH = plsc.VectorSubcoreMesh(
    core_axis_name="c", subcore_axis_name="s", num_cores=NC, num_subcores=NS
)


@functools.partial(
    pl.kernel,
    out_type=jax.ShapeDtypeStruct((BATCH, FEAT), jnp.float32),
    mesh=_MESH,
    scratch_types=[
        pltpu.VMEM((B_PER_W,), jnp.int32),
        pltpu.VMEM((N_CENTERS,), jnp.int32),
        pltpu.VMEM((B_PER_W,), jnp.int32),
        pltpu.VMEM((B_PER_W, FEAT), jnp.float32),
        pltpu.SemaphoreType.DMA,
    ],
)
def _sc_gather(table_hbm, ids_hbm, count_hbm, out_hbm, ids_v, cnt_v, gidx_v, rows_v, sem):
    wid = lax.axis_index("s") * NC + lax.axis_index("c")
    base = wid * B_PER_W
    pltpu.sync_copy(ids_hbm.at[pl.ds(base, B_PER_W)], ids_v)
    pltpu.sync_copy(count_hbm, cnt_v)

    lanes = lax.iota(jnp.int32, 16)

    def chunk(j, carry):
        cidx = ids_v[pl.ds(j * 16, 16)]
        cnt16 = plsc.load_gather(cnt_v, [cidx])
        fb_rows = (N_CENTERS + base + j * 16) + lanes
        gidx_v[pl.ds(j * 16, 16)] = jnp.where(cnt16 > 0, cidx, fb_rows)
        return carry

    lax.fori_loop(0, CHUNKS, chunk, 0)

    pltpu.async_copy(table_hbm.at[gidx_v], rows_v, sem).wait()
    pltpu.sync_copy(rows_v, out_hbm.at[pl.ds(base, B_PER_W)])


@jax.jit
def kernel(target_center_ids, bank, count):
    means = _center_means(count, bank)
    fallback = jax.random.normal(
        jax.random.key(1), (BATCH, FEAT), dtype=bank.dtype
    )
    table = jnp.concatenate([means, fallback], axis=0)
    return _sc_gather(table, target_center_ids, count)


# speedup vs baseline: 2.5093x; 2.5093x over previous
import functools

import jax
import jax.numpy as jnp
from jax import lax
from jax.experimental import pallas as pl
from jax.experimental.pallas import tpu as pltpu
from jax.experimental.pallas import tpu_sc as plsc

N_CENTERS = 10000
CAP = 100
FEAT = 64
BATCH = 16384


CB = 80
GRID = N_CENTERS // CB


def _means_body(cnt_ref, bank_ref, out_ref):
    cnt = cnt_ref[0, 0, :]
    rows = bank_ref[...]
    pos = lax.broadcasted_iota(jnp.int32, (CB, CAP, FEAT), 1)
    mask = (pos < cnt[:, None, None]).astype(jnp.float32)
    sums = jnp.sum(rows * mask, axis=1)
    denom = jnp.maximum(cnt, 1).astype(jnp.float32)[:, None]
    out_ref[...] = sums / denom


def _center_means(count, bank):
    cnt3 = count.reshape(GRID, 1, CB)
    return pl.pallas_call(
        _means_body,
        grid=(GRID,),
        in_specs=[
            pl.BlockSpec((1, 1, CB), lambda i: (i, 0, 0)),
            pl.BlockSpec((CB, CAP, FEAT), lambda i: (i, 0, 0)),
        ],
        out_specs=pl.BlockSpec((CB, FEAT), lambda i: (i, 0)),
        out_shape=jax.ShapeDtypeStruct((N_CENTERS, FEAT), jnp.float32),
    )(cnt3, bank)


NC = 2
NS = 16
NW = NC * NS
B_PER_W = BATCH // NW
CHUNKS = B_PER_W // 16

@functools.cache
def _build_sc_gather():
    mesh = plsc.VectorSubcoreMesh(
        core_axis_name="c", subcore_axis_name="s", num_cores=NC, num_subcores=NS
    )

    @functools.partial(
        pl.kernel,
        out_type=jax.ShapeDtypeStruct((BATCH, FEAT), jnp.float32),
        mesh=mesh,
        compiler_params=pltpu.CompilerParams(
            needs_layout_passes=False, use_tc_tiling_on_sc=False
        ),
        scratch_types=[
            pltpu.VMEM((B_PER_W,), jnp.int32),
            pltpu.VMEM((N_CENTERS,), jnp.int32),
            pltpu.VMEM((B_PER_W,), jnp.int32),
            pltpu.VMEM((B_PER_W, FEAT), jnp.float32),
            pltpu.SemaphoreType.DMA,
        ],
    )
    def _sc_gather(table_hbm, ids_hbm, count_hbm, out_hbm,
                   ids_v, cnt_v, gidx_v, rows_v, sem):
        wid = lax.axis_index("s") * NC + lax.axis_index("c")
        base = wid * B_PER_W
        pltpu.sync_copy(ids_hbm.at[pl.ds(base, B_PER_W)], ids_v)
        pltpu.sync_copy(count_hbm, cnt_v)

        lanes = lax.iota(jnp.int32, 16)

        def chunk(j, carry):
            cidx = ids_v[pl.ds(j * 16, 16)]
            cnt16 = plsc.load_gather(cnt_v, [cidx])
            fb_rows = (N_CENTERS + base + j * 16) + lanes
            gidx_v[pl.ds(j * 16, 16)] = jnp.where(cnt16 > 0, cidx, fb_rows)
            return carry

        lax.fori_loop(0, CHUNKS, chunk, 0)

        pltpu.async_copy(table_hbm.at[gidx_v], rows_v, sem).wait()
        pltpu.sync_copy(rows_v, out_hbm.at[pl.ds(base, B_PER_W)])

    return _sc_gather


@jax.jit
def kernel(target_center_ids, bank, count):
    means = _center_means(count, bank)
    fallback = jax.random.normal(
        jax.random.key(1), (BATCH, FEAT), dtype=bank.dtype
    )
    table = jnp.concatenate([means, fallback], axis=0)
    return _build_sc_gather()(table, target_center_ids, count)


# speedup vs baseline: 3.9983x; 1.5934x over previous
import functools

import jax
import jax.numpy as jnp
from jax import lax
from jax.experimental import pallas as pl
from jax.experimental.pallas import tpu as pltpu
from jax.experimental.pallas import tpu_sc as plsc

N_CENTERS = 10000
CAP = 100
FEAT = 64
BATCH = 16384
ROW = CAP * FEAT


CB = 200
GRID = N_CENTERS // CB


def _means_body(cnt_ref, rows_ref, grp_ref, e_ref, out_ref):
    cnt = cnt_ref[0, 0, :]
    rows = rows_ref[...]
    grp = grp_ref[...]
    mask = grp < cnt[:, None]
    masked = jnp.where(mask, rows, 0.0)
    sums = jnp.dot(masked, e_ref[...], preferred_element_type=jnp.float32)
    denom = jnp.maximum(cnt, 1).astype(jnp.float32)[:, None]
    out_ref[...] = sums / denom


def _center_means(count, bank2d, grp, e):
    cnt3 = count.reshape(GRID, 1, CB)
    return pl.pallas_call(
        _means_body,
        grid=(GRID,),
        in_specs=[
            pl.BlockSpec((1, 1, CB), lambda i: (i, 0, 0)),
            pl.BlockSpec((CB, ROW), lambda i: (i, 0)),
            pl.BlockSpec((1, ROW), lambda i: (0, 0)),
            pl.BlockSpec((ROW, FEAT), lambda i: (0, 0)),
        ],
        out_specs=pl.BlockSpec((CB, FEAT), lambda i: (i, 0)),
        out_shape=jax.ShapeDtypeStruct((N_CENTERS, FEAT), jnp.float32),
    )(cnt3, bank2d, grp, e)


NC = 2
NS = 16
NW = NC * NS
B_PER_W = BATCH // NW
CHUNKS = B_PER_W // 16


@functools.cache
def _build_sc_gather():
    mesh = plsc.VectorSubcoreMesh(
        core_axis_name="c", subcore_axis_name="s", num_cores=NC, num_subcores=NS
    )

    @functools.partial(
        pl.kernel,
        out_type=jax.ShapeDtypeStruct((BATCH, FEAT), jnp.float32),
        mesh=mesh,
        compiler_params=pltpu.CompilerParams(
            needs_layout_passes=False, use_tc_tiling_on_sc=False
        ),
        scratch_types=[
            pltpu.VMEM((B_PER_W,), jnp.int32),
            pltpu.VMEM((N_CENTERS,), jnp.int32),
            pltpu.VMEM((B_PER_W,), jnp.int32),
            pltpu.VMEM((B_PER_W, FEAT), jnp.float32),
            pltpu.SemaphoreType.DMA,
        ],
    )
    def _sc_gather(table_hbm, ids_hbm, count_hbm, out_hbm,
                   ids_v, cnt_v, gidx_v, rows_v, sem):
        wid = lax.axis_index("s") * NC + lax.axis_index("c")
        base = wid * B_PER_W
        pltpu.sync_copy(ids_hbm.at[pl.ds(base, B_PER_W)], ids_v)
        pltpu.sync_copy(count_hbm, cnt_v)

        lanes = lax.iota(jnp.int32, 16)

        def chunk(j, carry):
            cidx = ids_v[pl.ds(j * 16, 16)]
            cnt16 = plsc.load_gather(cnt_v, [cidx])
            fb_rows = (N_CENTERS + base + j * 16) + lanes
            gidx_v[pl.ds(j * 16, 16)] = jnp.where(cnt16 > 0, cidx, fb_rows)
            return carry

        lax.fori_loop(0, CHUNKS, chunk, 0)

        pltpu.async_copy(table_hbm.at[gidx_v], rows_v, sem).wait()
        pltpu.sync_copy(rows_v, out_hbm.at[pl.ds(base, B_PER_W)])

    return _sc_gather


@functools.cache
def _consts():
    fallback = jax.random.normal(
        jax.random.key(1), (BATCH, FEAT), dtype=jnp.float32
    )
    grp = (jnp.arange(ROW, dtype=jnp.int32) // FEAT)[None, :]
    e = (
        (jnp.arange(ROW, dtype=jnp.int32)[:, None] % FEAT)
        == jnp.arange(FEAT, dtype=jnp.int32)[None, :]
    ).astype(jnp.float32)
    return (
        jax.block_until_ready(fallback),
        jax.block_until_ready(grp),
        jax.block_until_ready(e),
    )


@jax.jit
def _run(target_center_ids, bank, count, fallback, grp, e):
    bank2d = bank.reshape(N_CENTERS, ROW)
    means = _center_means(count, bank2d, grp, e)
    table = jnp.concatenate([means, fallback], axis=0)
    return _build_sc_gather()(table, target_center_ids, count)


def kernel(target_center_ids, bank, count):
    fallback, grp, e = _consts()
    return _run(target_center_ids, bank, count, fallback, grp, e)
